# baseline (speedup 1.0000x reference)
import functools

import jax
import jax.numpy as jnp
from jax.experimental import pallas as pl

IN_F = 2048
OUT_F = 2048
RDIM = 32
KSEL = 8
SCALE = 64.0 / 32.0


def _fused_kernel(x_ref, a_ref, b_ref, d_ref, o_ref):
    x_blk = x_ref[...]
    a = a_ref[...]
    b = b_ref[...]
    d = d_ref[...]

    y = jax.lax.dot_general(
        x_blk, a, (((1,), (1,)), ((), ())),
        preferred_element_type=jnp.float32)
    yb = jnp.abs(y + d)

    bl = y.shape[0]
    lane = jax.lax.broadcasted_iota(jnp.int32, (bl, RDIM), 1)
    rank = jnp.zeros((bl, RDIM), jnp.int32)
    for j in range(RDIM):
        colj = yb[:, j:j + 1]
        beats = (colj > yb) | ((colj == yb) & (j < lane))
        rank = rank + beats.astype(jnp.int32)
    mask = (rank < KSEL).astype(jnp.float32)

    act = y * mask
    out = jax.lax.dot_general(
        act, b, (((1,), (1,)), ((), ())),
        preferred_element_type=jnp.float32)
    o_ref[...] = out * SCALE


@jax.jit
def kernel(x, A, B, d):
    n_tokens = x.shape[0]
    bl = 512
    grid = (n_tokens // bl,)
    d2 = d.reshape(1, RDIM)
    return pl.pallas_call(
        _fused_kernel,
        grid=grid,
        in_specs=[
            pl.BlockSpec((bl, IN_F), lambda i: (i, 0)),
            pl.BlockSpec((RDIM, IN_F), lambda i: (0, 0)),
            pl.BlockSpec((OUT_F, RDIM), lambda i: (0, 0)),
            pl.BlockSpec((1, RDIM), lambda i: (0, 0)),
        ],
        out_specs=pl.BlockSpec((bl, OUT_F), lambda i: (i, 0)),
        out_shape=jax.ShapeDtypeStruct((n_tokens, OUT_F), jnp.float32),
    )(x, A, B, d2)


# speedup vs baseline: 1.8392x; 1.8392x over previous
import jax
import jax.numpy as jnp
from jax.experimental import pallas as pl
from jax.experimental.pallas import tpu as pltpu

IN_F = 2048
OUT_F = 2048
RDIM = 32
KSEL = 8
SCALE = 64.0 / 32.0


def _fused_kernel(x_ref, a_ref, b_ref, d_ref, o_ref):
    x_blk = x_ref[...]
    a = a_ref[...]
    b = b_ref[...]
    d = d_ref[...]

    y = jax.lax.dot_general(
        x_blk, a, (((1,), (1,)), ((), ())),
        preferred_element_type=jnp.float32)
    yb = jnp.abs(y + d)

    keys = jnp.transpose(yb).view(jnp.int32)
    row = jax.lax.broadcasted_iota(jnp.int32, (RDIM, keys.shape[1]), 0)
    rank = jnp.zeros(keys.shape, jnp.int32)
    for j in range(RDIM):
        kj = jnp.zeros_like(keys) + keys[j:j + 1, :]
        gt = (kj > keys).astype(jnp.int32)
        ge = (kj >= keys).astype(jnp.int32)
        rank = rank + jnp.where(row > j, ge, gt)
    mask_t = jnp.where(rank < KSEL, jnp.float32(SCALE), jnp.float32(0.0))
    mask = jnp.transpose(mask_t)

    act = (y * mask).astype(jnp.bfloat16)
    out = jax.lax.dot_general(
        act, b, (((1,), (1,)), ((), ())),
        preferred_element_type=jnp.float32)
    o_ref[...] = out


@jax.jit
def kernel(x, A, B, d):
    n_tokens = x.shape[0]
    bl = 512
    grid = (n_tokens // bl,)
    d2 = d.reshape(1, RDIM)
    b_bf = B.astype(jnp.bfloat16)
    return pl.pallas_call(
        _fused_kernel,
        grid=grid,
        in_specs=[
            pl.BlockSpec((bl, IN_F), lambda i: (i, 0)),
            pl.BlockSpec((RDIM, IN_F), lambda i: (0, 0)),
            pl.BlockSpec((OUT_F, RDIM), lambda i: (0, 0)),
            pl.BlockSpec((1, RDIM), lambda i: (0, 0)),
        ],
        out_specs=pl.BlockSpec((bl, OUT_F), lambda i: (i, 0)),
        out_shape=jax.ShapeDtypeStruct((n_tokens, OUT_F), jnp.float32),
        compiler_params=pltpu.CompilerParams(
            dimension_semantics=("parallel",)),
    )(x, A, b_bf, d2)


# speedup vs baseline: 1.9849x; 1.0792x over previous
import jax
import jax.numpy as jnp
from jax.experimental import pallas as pl
from jax.experimental.pallas import tpu as pltpu

IN_F = 2048
OUT_F = 2048
RDIM = 32
KSEL = 8
SCALE = 64.0 / 32.0


def _fused_kernel(x_ref, a_ref, b_ref, d_ref, o_ref):
    x_blk = x_ref[...]
    a = a_ref[...]
    b = b_ref[...]
    d = d_ref[...]

    y = jax.lax.dot_general(
        x_blk, a, (((1,), (1,)), ((), ())),
        preferred_element_type=jnp.float32)
    yb = jnp.abs(y + d)

    keys = jnp.transpose(yb).view(jnp.int32)
    row = jax.lax.broadcasted_iota(jnp.int32, (RDIM, keys.shape[1]), 0)
    rank = jnp.zeros(keys.shape, jnp.int32)
    for j in range(RDIM):
        kj = jnp.zeros_like(keys) + keys[j:j + 1, :]
        gt = (kj > keys).astype(jnp.int32)
        ge = (kj >= keys).astype(jnp.int32)
        rank = rank + jnp.where(row > j, ge, gt)
    mask_t = jnp.where(rank < KSEL, jnp.float32(SCALE), jnp.float32(0.0))
    mask = jnp.transpose(mask_t)

    act = (y * mask).astype(jnp.bfloat16)
    out = jax.lax.dot_general(
        act, b, (((1,), (1,)), ((), ())),
        preferred_element_type=jnp.float32)
    o_ref[...] = out


@jax.jit
def kernel(x, A, B, d):
    n_tokens = x.shape[0]
    bl = 1024
    grid = (n_tokens // bl,)
    d2 = d.reshape(1, RDIM)
    b_bf = B.astype(jnp.bfloat16)
    return pl.pallas_call(
        _fused_kernel,
        grid=grid,
        in_specs=[
            pl.BlockSpec((bl, IN_F), lambda i: (i, 0)),
            pl.BlockSpec((RDIM, IN_F), lambda i: (0, 0)),
            pl.BlockSpec((OUT_F, RDIM), lambda i: (0, 0)),
            pl.BlockSpec((1, RDIM), lambda i: (0, 0)),
        ],
        out_specs=pl.BlockSpec((bl, OUT_F), lambda i: (i, 0)),
        out_shape=jax.ShapeDtypeStruct((n_tokens, OUT_F), jnp.float32),
        compiler_params=pltpu.CompilerParams(
            dimension_semantics=("parallel",)),
    )(x, A, b_bf, d2)
